# baseline (speedup 1.0000x reference)
import jax
import jax.numpy as jnp
from jax import lax
from jax.experimental import pallas as pl
from jax.experimental.pallas import tpu as pltpu
from jax.experimental.pallas import tpu_sc as plsc

B = 1024
T = 200
C = 64
N = B * T

NC = 2
NS = 16
NW = NC * NS
L = 16

ROWS_PER_W = N // NW
CHUNK = 128
NCHUNK = ROWS_PER_W // CHUNK
NB = 2
PAIRS = NCHUNK // NB

EPS = 1e-5


def _rsqrt(v):
    i = lax.bitcast_convert_type(v, jnp.int32)
    y = lax.bitcast_convert_type(
        jnp.int32(0x5F3759DF) - lax.shift_right_arithmetic(i, 1), jnp.float32)
    half_v = v * 0.5
    for _ in range(2):
        y = y * (1.5 - half_v * y * y)
    return y


def _body(x_hbm, temb_hbm, pemb_hbm, gamma_hbm, beta_hbm, out_hbm,
          idx_all, rows_v, out_v, pemb_v, g_v, b_v,
          gsem0, gsem1, wsem0, wsem1):
    gsem = (gsem0, gsem1)
    wsem = (wsem0, wsem1)
    wid = lax.axis_index("s") * NC + lax.axis_index("c")
    base = wid * ROWS_PER_W

    pltpu.sync_copy(pemb_hbm, pemb_v.at[pl.ds(0, T)])
    pltpu.sync_copy(pemb_hbm, pemb_v.at[pl.ds(T, T)])
    pltpu.sync_copy(gamma_hbm, g_v)
    pltpu.sync_copy(beta_hbm, b_v)
    pltpu.sync_copy(x_hbm.at[pl.ds(pl.multiple_of(base, ROWS_PER_W),
                                   ROWS_PER_W)], idx_all)

    gv = [g_v[pl.ds(c * L, L)] for c in range(C // L)]
    bv = [b_v[pl.ds(c * L, L)] for c in range(C // L)]

    def gather_start(c, b):
        off = pl.multiple_of(c * CHUNK, CHUNK)
        return pltpu.async_copy(
            temb_hbm.at[idx_all.at[pl.ds(off, CHUNK)]], rows_v.at[b],
            gsem[b])

    def write_desc(c, b):
        off = pl.multiple_of(base + c * CHUNK, CHUNK)
        return pltpu.make_async_copy(
            out_v.at[b], out_hbm.at[pl.ds(off, CHUNK)], wsem[b])

    gather_start(0, 0)
    gather_start(1, 1)

    def pair_body(m, carry):
        for b in range(NB):
            c = m * NB + b
            off = pl.multiple_of(c * CHUNK, CHUNK)
            pltpu.make_async_copy(
                temb_hbm.at[idx_all.at[pl.ds(off, CHUNK)]], rows_v.at[b],
                gsem[b]).wait()

            @pl.when(m > 0)
            def _wait_prev():
                write_desc(c - NB, b).wait()

            t0 = lax.rem(c * CHUNK, T)

            def row_body(r, carry):
                t = t0 + r
                h = [rows_v[b, r, pl.ds(ci * L, L)]
                     + pemb_v[t, pl.ds(ci * L, L)]
                     for ci in range(C // L)]
                s = (h[0] + h[1]) + (h[2] + h[3])
                sq = ((h[0] * h[0] + h[1] * h[1])
                      + (h[2] * h[2] + h[3] * h[3]))
                mean = jnp.sum(s) * (1.0 / C)
                ms2 = jnp.sum(sq) * (1.0 / C)
                var = ms2 - mean * mean
                rstd = _rsqrt(var + EPS)
                mr_v = jnp.broadcast_to(mean * rstd, (L,))
                rstd_v = jnp.broadcast_to(rstd, (L,))
                for ci in range(C // L):
                    hn = h[ci] * rstd_v - mr_v
                    out_v[b, r, pl.ds(ci * L, L)] = hn * gv[ci] + bv[ci]
                return carry

            lax.fori_loop(0, CHUNK, row_body, jnp.int32(0), unroll=4)

            woff = pl.multiple_of(base + c * CHUNK, CHUNK)
            pltpu.async_copy(out_v.at[b],
                             out_hbm.at[pl.ds(woff, CHUNK)], wsem[b])

            @pl.when(c + NB < NCHUNK)
            def _prefetch():
                gather_start(c + NB, b)
        return carry

    lax.fori_loop(0, PAIRS, pair_body, jnp.int32(0))

    for b in range(NB):
        write_desc(NCHUNK - NB + b, b).wait()


def _run(x1, temb, pemb, gamma, beta):
    mesh = plsc.VectorSubcoreMesh(
        core_axis_name="c", subcore_axis_name="s",
        num_cores=NC, num_subcores=NS)
    f = pl.kernel(
        _body,
        out_type=jax.ShapeDtypeStruct((N, C), jnp.float32),
        mesh=mesh,
        scratch_types=[
            pltpu.VMEM((ROWS_PER_W,), jnp.int32),
            pltpu.VMEM((NB, CHUNK, C), jnp.float32),
            pltpu.VMEM((NB, CHUNK, C), jnp.float32),
            pltpu.VMEM((2 * T, C), jnp.float32),
            pltpu.VMEM((C,), jnp.float32),
            pltpu.VMEM((C,), jnp.float32),
            pltpu.SemaphoreType.DMA,
            pltpu.SemaphoreType.DMA,
            pltpu.SemaphoreType.DMA,
            pltpu.SemaphoreType.DMA,
        ],
        compiler_params=pltpu.CompilerParams(
            needs_layout_passes=False, use_tc_tiling_on_sc=False),
    )
    return f(x1, temb, pemb, gamma, beta)


@jax.jit
def _kernel_impl(x, temb, pemb, gamma, beta):
    out = _run(x.reshape(N), temb, pemb, gamma, beta)
    return out.reshape(B, T, C)


def kernel(x, temb, pemb, gamma, beta):
    return _kernel_impl(x, temb, pemb, gamma, beta)


# speedup vs baseline: 1.1143x; 1.1143x over previous
import jax
import jax.numpy as jnp
from jax import lax
from jax.experimental import pallas as pl
from jax.experimental.pallas import tpu as pltpu
from jax.experimental.pallas import tpu_sc as plsc

B = 1024
T = 200
C = 64
N = B * T

NC = 2
NS = 16
NW = NC * NS
L = 16

ROWS_PER_W = N // NW
CHUNK = 128
NCHUNK = ROWS_PER_W // CHUNK
NB = 2
PAIRS = NCHUNK // NB

EPS = 1e-5


def _rsqrt(v):
    i = lax.bitcast_convert_type(v, jnp.int32)
    y = lax.bitcast_convert_type(
        jnp.int32(0x5F3759DF) - lax.shift_right_arithmetic(i, 1), jnp.float32)
    half_v = v * 0.5
    for _ in range(2):
        y = y * (1.5 - half_v * y * y)
    return y


def _body(x_hbm, temb_hbm, pemb_hbm, gamma_hbm, beta_hbm, out_hbm,
          idx_all, rows_v, out_v, pemb_v, g_v, b_v,
          gsem0, gsem1, wsem0, wsem1):
    gsem = (gsem0, gsem1)
    wsem = (wsem0, wsem1)
    wid = lax.axis_index("s") * NC + lax.axis_index("c")
    base = wid * ROWS_PER_W

    pltpu.sync_copy(pemb_hbm, pemb_v.at[pl.ds(0, T)])
    pltpu.sync_copy(pemb_hbm, pemb_v.at[pl.ds(T, T)])
    pltpu.sync_copy(gamma_hbm, g_v)
    pltpu.sync_copy(beta_hbm, b_v)
    pltpu.sync_copy(x_hbm.at[pl.ds(pl.multiple_of(base, ROWS_PER_W),
                                   ROWS_PER_W)], idx_all)

    gv = [g_v[pl.ds(c * L, L)] for c in range(C // L)]
    bv = [b_v[pl.ds(c * L, L)] for c in range(C // L)]

    def gather_start(c, b):
        off = pl.multiple_of(c * CHUNK, CHUNK)
        return pltpu.async_copy(
            temb_hbm.at[idx_all.at[pl.ds(off, CHUNK)]], rows_v.at[b],
            gsem[b])

    def write_desc(c, b):
        off = pl.multiple_of(base + c * CHUNK, CHUNK)
        return pltpu.make_async_copy(
            out_v.at[b], out_hbm.at[pl.ds(off, CHUNK)], wsem[b])

    gather_start(0, 0)
    gather_start(1, 1)

    def pair_body(m, carry):
        for b in range(NB):
            c = m * NB + b
            off = pl.multiple_of(c * CHUNK, CHUNK)
            pltpu.make_async_copy(
                temb_hbm.at[idx_all.at[pl.ds(off, CHUNK)]], rows_v.at[b],
                gsem[b]).wait()

            @pl.when(m > 0)
            def _wait_prev():
                write_desc(c - NB, b).wait()

            t0 = lax.rem(c * CHUNK, T)

            def row_body(r, carry):
                t = t0 + r
                h = [rows_v[b, r, pl.ds(ci * L, L)]
                     + pemb_v[t, pl.ds(ci * L, L)]
                     for ci in range(C // L)]
                s = (h[0] + h[1]) + (h[2] + h[3])
                sq = ((h[0] * h[0] + h[1] * h[1])
                      + (h[2] * h[2] + h[3] * h[3]))
                mean = jnp.broadcast_to(jnp.sum(s), (L,)) * (1.0 / C)
                ms2 = jnp.broadcast_to(jnp.sum(sq), (L,)) * (1.0 / C)
                var = ms2 - mean * mean
                rstd = _rsqrt(var + EPS)
                mr = mean * rstd
                for ci in range(C // L):
                    hn = h[ci] * rstd - mr
                    out_v[b, r, pl.ds(ci * L, L)] = hn * gv[ci] + bv[ci]
                return carry

            lax.fori_loop(0, CHUNK, row_body, jnp.int32(0), unroll=4)

            woff = pl.multiple_of(base + c * CHUNK, CHUNK)
            pltpu.async_copy(out_v.at[b],
                             out_hbm.at[pl.ds(woff, CHUNK)], wsem[b])

            @pl.when(c + NB < NCHUNK)
            def _prefetch():
                gather_start(c + NB, b)
        return carry

    lax.fori_loop(0, PAIRS, pair_body, jnp.int32(0))

    for b in range(NB):
        write_desc(NCHUNK - NB + b, b).wait()


def _run(x1, temb, pemb, gamma, beta):
    mesh = plsc.VectorSubcoreMesh(
        core_axis_name="c", subcore_axis_name="s",
        num_cores=NC, num_subcores=NS)
    f = pl.kernel(
        _body,
        out_type=jax.ShapeDtypeStruct((N, C), jnp.float32),
        mesh=mesh,
        scratch_types=[
            pltpu.VMEM((ROWS_PER_W,), jnp.int32),
            pltpu.VMEM((NB, CHUNK, C), jnp.float32),
            pltpu.VMEM((NB, CHUNK, C), jnp.float32),
            pltpu.VMEM((2 * T, C), jnp.float32),
            pltpu.VMEM((C,), jnp.float32),
            pltpu.VMEM((C,), jnp.float32),
            pltpu.SemaphoreType.DMA,
            pltpu.SemaphoreType.DMA,
            pltpu.SemaphoreType.DMA,
            pltpu.SemaphoreType.DMA,
        ],
        compiler_params=pltpu.CompilerParams(
            needs_layout_passes=False, use_tc_tiling_on_sc=False),
    )
    return f(x1, temb, pemb, gamma, beta)


@jax.jit
def _kernel_impl(x, temb, pemb, gamma, beta):
    out = _run(x.reshape(N), temb, pemb, gamma, beta)
    return out.reshape(B, T, C)


def kernel(x, temb, pemb, gamma, beta):
    return _kernel_impl(x, temb, pemb, gamma, beta)


# speedup vs baseline: 1.1148x; 1.0004x over previous
import jax
import jax.numpy as jnp
from jax import lax
from jax.experimental import pallas as pl
from jax.experimental.pallas import tpu as pltpu
from jax.experimental.pallas import tpu_sc as plsc

B = 1024
T = 200
C = 64
N = B * T

NC = 2
NS = 16
NW = NC * NS
L = 16

ROWS_PER_W = N // NW
CHUNK = 128
NCHUNK = ROWS_PER_W // CHUNK
NB = 2
PAIRS = NCHUNK // NB

EPS = 1e-5


def _rsqrt(v):
    i = lax.bitcast_convert_type(v, jnp.int32)
    y = lax.bitcast_convert_type(
        jnp.int32(0x5F3759DF) - lax.shift_right_arithmetic(i, 1), jnp.float32)
    half_v = v * 0.5
    for _ in range(2):
        y = y * (1.5 - half_v * y * y)
    return y


def _body(x_hbm, temb_hbm, pemb_hbm, gamma_hbm, beta_hbm, out_hbm,
          idx_all, rows_v, out_v, pemb_v, g_v, b_v,
          gsem0, gsem1, wsem0, wsem1):
    gsem = (gsem0, gsem1)
    wsem = (wsem0, wsem1)
    wid = lax.axis_index("s") * NC + lax.axis_index("c")
    base = wid * ROWS_PER_W

    pltpu.sync_copy(pemb_hbm, pemb_v.at[pl.ds(0, T)])
    pltpu.sync_copy(pemb_hbm, pemb_v.at[pl.ds(T, T)])
    pltpu.sync_copy(gamma_hbm, g_v)
    pltpu.sync_copy(beta_hbm, b_v)
    pltpu.sync_copy(x_hbm.at[pl.ds(pl.multiple_of(base, ROWS_PER_W),
                                   ROWS_PER_W)], idx_all)

    gv = [g_v[pl.ds(c * L, L)] for c in range(C // L)]
    bv = [b_v[pl.ds(c * L, L)] for c in range(C // L)]

    def gather_start(c, b):
        off = pl.multiple_of(c * CHUNK, CHUNK)
        return pltpu.async_copy(
            temb_hbm.at[idx_all.at[pl.ds(off, CHUNK)]], rows_v.at[b],
            gsem[b])

    def write_desc(c, b):
        off = pl.multiple_of(base + c * CHUNK, CHUNK)
        return pltpu.make_async_copy(
            out_v.at[b], out_hbm.at[pl.ds(off, CHUNK)], wsem[b])

    gather_start(0, 0)
    gather_start(1, 1)

    def pair_body(m, carry):
        for b in range(NB):
            c = m * NB + b
            off = pl.multiple_of(c * CHUNK, CHUNK)
            pltpu.make_async_copy(
                temb_hbm.at[idx_all.at[pl.ds(off, CHUNK)]], rows_v.at[b],
                gsem[b]).wait()

            @pl.when(m > 0)
            def _wait_prev():
                write_desc(c - NB, b).wait()

            t0 = lax.rem(c * CHUNK, T)

            def row_body(r, carry):
                t = t0 + r
                h = [rows_v[b, r, pl.ds(ci * L, L)]
                     + pemb_v[t, pl.ds(ci * L, L)]
                     for ci in range(C // L)]
                s = (h[0] + h[1]) + (h[2] + h[3])
                sq = ((h[0] * h[0] + h[1] * h[1])
                      + (h[2] * h[2] + h[3] * h[3]))
                mean = jnp.broadcast_to(jnp.sum(s), (L,)) * (1.0 / C)
                ms2 = jnp.broadcast_to(jnp.sum(sq), (L,)) * (1.0 / C)
                var = ms2 - mean * mean
                rstd = _rsqrt(var + EPS)
                mr = mean * rstd
                for ci in range(C // L):
                    hn = h[ci] * rstd - mr
                    out_v[b, r, pl.ds(ci * L, L)] = hn * gv[ci] + bv[ci]
                return carry

            lax.fori_loop(0, CHUNK, row_body, jnp.int32(0), unroll=8)

            woff = pl.multiple_of(base + c * CHUNK, CHUNK)
            pltpu.async_copy(out_v.at[b],
                             out_hbm.at[pl.ds(woff, CHUNK)], wsem[b])

            @pl.when(c + NB < NCHUNK)
            def _prefetch():
                gather_start(c + NB, b)
        return carry

    lax.fori_loop(0, PAIRS, pair_body, jnp.int32(0))

    for b in range(NB):
        write_desc(NCHUNK - NB + b, b).wait()


def _run(x1, temb, pemb, gamma, beta):
    mesh = plsc.VectorSubcoreMesh(
        core_axis_name="c", subcore_axis_name="s",
        num_cores=NC, num_subcores=NS)
    f = pl.kernel(
        _body,
        out_type=jax.ShapeDtypeStruct((N, C), jnp.float32),
        mesh=mesh,
        scratch_types=[
            pltpu.VMEM((ROWS_PER_W,), jnp.int32),
            pltpu.VMEM((NB, CHUNK, C), jnp.float32),
            pltpu.VMEM((NB, CHUNK, C), jnp.float32),
            pltpu.VMEM((2 * T, C), jnp.float32),
            pltpu.VMEM((C,), jnp.float32),
            pltpu.VMEM((C,), jnp.float32),
            pltpu.SemaphoreType.DMA,
            pltpu.SemaphoreType.DMA,
            pltpu.SemaphoreType.DMA,
            pltpu.SemaphoreType.DMA,
        ],
        compiler_params=pltpu.CompilerParams(
            needs_layout_passes=False, use_tc_tiling_on_sc=False),
    )
    return f(x1, temb, pemb, gamma, beta)


@jax.jit
def _kernel_impl(x, temb, pemb, gamma, beta):
    out = _run(x.reshape(N), temb, pemb, gamma, beta)
    return out.reshape(B, T, C)


def kernel(x, temb, pemb, gamma, beta):
    return _kernel_impl(x, temb, pemb, gamma, beta)


# speedup vs baseline: 1.6424x; 1.4732x over previous
import jax
import jax.numpy as jnp
from jax import lax
from jax.experimental import pallas as pl
from jax.experimental.pallas import tpu as pltpu
from jax.experimental.pallas import tpu_sc as plsc

B = 1024
T = 200
C = 64
N = B * T

NC = 2
NS = 16
NW = NC * NS
L = 16

ROWS_PER_W = N // NW
CHUNK = 128
NCHUNK = ROWS_PER_W // CHUNK
NB = 2
PAIRS = NCHUNK // NB

EPS = 1e-5


def _rsqrt(v):
    i = lax.bitcast_convert_type(v, jnp.int32)
    y = lax.bitcast_convert_type(
        jnp.int32(0x5F3759DF) - lax.shift_right_arithmetic(i, 1), jnp.float32)
    half_v = v * 0.5
    for _ in range(2):
        y = y * (1.5 - half_v * y * y)
    return y


def _body(x_hbm, temb_hbm, pemb_hbm, gamma_hbm, beta_hbm, out_hbm,
          idx_all, rows_v, out_v, pemb_v, g_v, b_v,
          gsem0, gsem1, wsem0, wsem1):
    gsem = (gsem0, gsem1)
    wsem = (wsem0, wsem1)
    wid = lax.axis_index("s") * NC + lax.axis_index("c")
    base = wid * ROWS_PER_W

    pltpu.sync_copy(pemb_hbm, pemb_v.at[pl.ds(0, T)])
    pltpu.sync_copy(pemb_hbm, pemb_v.at[pl.ds(T, T)])
    pltpu.sync_copy(gamma_hbm, g_v)
    pltpu.sync_copy(beta_hbm, b_v)
    pltpu.sync_copy(x_hbm.at[pl.ds(pl.multiple_of(base, ROWS_PER_W),
                                   ROWS_PER_W)], idx_all)

    gv = [g_v[pl.ds(c * L, L)] for c in range(C // L)]
    bv = [b_v[pl.ds(c * L, L)] for c in range(C // L)]

    def gather_start(c, b):
        off = pl.multiple_of(c * CHUNK, CHUNK)
        return pltpu.async_copy(
            temb_hbm.at[idx_all.at[pl.ds(off, CHUNK)]], rows_v.at[b],
            gsem[b])

    def write_desc(c, b):
        off = pl.multiple_of(base + c * CHUNK, CHUNK)
        return pltpu.make_async_copy(
            out_v.at[b], out_hbm.at[pl.ds(off, CHUNK)], wsem[b])

    gather_start(0, 0)
    gather_start(1, 1)

    def pair_body(m, carry):
        for b in range(NB):
            c = m * NB + b
            off = pl.multiple_of(c * CHUNK, CHUNK)
            pltpu.make_async_copy(
                temb_hbm.at[idx_all.at[pl.ds(off, CHUNK)]], rows_v.at[b],
                gsem[b]).wait()

            @pl.when(m > 0)
            def _wait_prev():
                write_desc(c - NB, b).wait()

            t0 = lax.rem(c * CHUNK, T)

            RB = 4

            def blk_body(m, carry):
                r0 = m * RB
                h = [[rows_v[b, r0 + i, pl.ds(ci * L, L)]
                      + pemb_v[t0 + r0 + i, pl.ds(ci * L, L)]
                      for ci in range(C // L)]
                     for i in range(RB)]
                s = [(hh[0] + hh[1]) + (hh[2] + hh[3]) for hh in h]
                sq = [((hh[0] * hh[0] + hh[1] * hh[1])
                       + (hh[2] * hh[2] + hh[3] * hh[3])) for hh in h]
                mean = [jnp.broadcast_to(jnp.sum(x), (L,)) * (1.0 / C)
                        for x in s]
                ms2 = [jnp.broadcast_to(jnp.sum(x), (L,)) * (1.0 / C)
                       for x in sq]
                var = [m2 - mu * mu for m2, mu in zip(ms2, mean)]
                rstd = [_rsqrt(v + EPS) for v in var]
                mr = [mu * rs for mu, rs in zip(mean, rstd)]
                for i in range(RB):
                    for ci in range(C // L):
                        hn = h[i][ci] * rstd[i] - mr[i]
                        out_v[b, r0 + i, pl.ds(ci * L, L)] = (
                            hn * gv[ci] + bv[ci])
                return carry

            lax.fori_loop(0, CHUNK // RB, blk_body, jnp.int32(0))

            woff = pl.multiple_of(base + c * CHUNK, CHUNK)
            pltpu.async_copy(out_v.at[b],
                             out_hbm.at[pl.ds(woff, CHUNK)], wsem[b])

            @pl.when(c + NB < NCHUNK)
            def _prefetch():
                gather_start(c + NB, b)
        return carry

    lax.fori_loop(0, PAIRS, pair_body, jnp.int32(0))

    for b in range(NB):
        write_desc(NCHUNK - NB + b, b).wait()


def _run(x1, temb, pemb, gamma, beta):
    mesh = plsc.VectorSubcoreMesh(
        core_axis_name="c", subcore_axis_name="s",
        num_cores=NC, num_subcores=NS)
    f = pl.kernel(
        _body,
        out_type=jax.ShapeDtypeStruct((N, C), jnp.float32),
        mesh=mesh,
        scratch_types=[
            pltpu.VMEM((ROWS_PER_W,), jnp.int32),
            pltpu.VMEM((NB, CHUNK, C), jnp.float32),
            pltpu.VMEM((NB, CHUNK, C), jnp.float32),
            pltpu.VMEM((2 * T, C), jnp.float32),
            pltpu.VMEM((C,), jnp.float32),
            pltpu.VMEM((C,), jnp.float32),
            pltpu.SemaphoreType.DMA,
            pltpu.SemaphoreType.DMA,
            pltpu.SemaphoreType.DMA,
            pltpu.SemaphoreType.DMA,
        ],
        compiler_params=pltpu.CompilerParams(
            needs_layout_passes=False, use_tc_tiling_on_sc=False),
    )
    return f(x1, temb, pemb, gamma, beta)


@jax.jit
def _kernel_impl(x, temb, pemb, gamma, beta):
    out = _run(x.reshape(N), temb, pemb, gamma, beta)
    return out.reshape(B, T, C)


def kernel(x, temb, pemb, gamma, beta):
    return _kernel_impl(x, temb, pemb, gamma, beta)


# speedup vs baseline: 1.9105x; 1.1633x over previous
import jax
import jax.numpy as jnp
from jax import lax
from jax.experimental import pallas as pl
from jax.experimental.pallas import tpu as pltpu
from jax.experimental.pallas import tpu_sc as plsc

B = 1024
T = 200
C = 64
N = B * T

NC = 2
NS = 16
NW = NC * NS
L = 16

B_PER_W = B // NW
ROWS_PER_W = N // NW
CHUNK = T
NCHUNK = B_PER_W
NB = 2
PAIRS = NCHUNK // NB
W0 = 128
W1 = CHUNK - W0

EPS = 1e-5


def _rsqrt(v):
    i = lax.bitcast_convert_type(v, jnp.int32)
    y = lax.bitcast_convert_type(
        jnp.int32(0x5F3759DF) - lax.shift_right_arithmetic(i, 1), jnp.float32)
    half_v = v * 0.5
    for _ in range(2):
        y = y * (1.5 - half_v * y * y)
    return y


def _body(x_hbm, temb_hbm, pemb_hbm, gamma_hbm, beta_hbm, out_hbm,
          idx_all, rows_v, out_v, pemb_v, g_v, b_v,
          gsem0, gsem1, wsem0, wsem1):
    gsem = (gsem0, gsem1)
    wsem = (wsem0, wsem1)
    wid = lax.axis_index("s") * NC + lax.axis_index("c")
    base = wid * ROWS_PER_W
    base_b = wid * B_PER_W

    pltpu.sync_copy(pemb_hbm, pemb_v)
    pltpu.sync_copy(gamma_hbm, g_v)
    pltpu.sync_copy(beta_hbm, b_v)
    pltpu.sync_copy(x_hbm.at[pl.ds(pl.multiple_of(base, ROWS_PER_W),
                                   ROWS_PER_W)], idx_all)

    gv = [g_v[pl.ds(c * L, L)] for c in range(C // L)]
    bv = [b_v[pl.ds(c * L, L)] for c in range(C // L)]

    def gather_descs(c, b, make_only=False):
        off = pl.multiple_of(c * CHUNK, 8)
        mk = pltpu.make_async_copy if make_only else pltpu.async_copy
        d0 = mk(temb_hbm.at[idx_all.at[pl.ds(off, W0)]],
                rows_v.at[b, pl.ds(0, W0)], gsem[b])
        d1 = mk(temb_hbm.at[idx_all.at[pl.ds(off + W0, W1)]],
                rows_v.at[b, pl.ds(W0, W1)], gsem[b])
        return d0, d1

    def write_desc(c, b):
        return pltpu.make_async_copy(out_v.at[b], out_hbm.at[base_b + c],
                                     wsem[b])

    gather_descs(0, 0)
    gather_descs(1, 1)

    def pair_body(m, carry):
        for b in range(NB):
            c = m * NB + b
            for d in gather_descs(c, b, make_only=True):
                d.wait()

            @pl.when(m > 0)
            def _wait_prev():
                write_desc(c - NB, b).wait()

            RB = 4

            def blk_body(mm, carry2):
                r0 = mm * RB
                h = [[rows_v[b, r0 + i, pl.ds(ci * L, L)]
                      + pemb_v[r0 + i, pl.ds(ci * L, L)]
                      for ci in range(C // L)]
                     for i in range(RB)]
                s = [(hh[0] + hh[1]) + (hh[2] + hh[3]) for hh in h]
                sq = [((hh[0] * hh[0] + hh[1] * hh[1])
                       + (hh[2] * hh[2] + hh[3] * hh[3])) for hh in h]
                mean = [jnp.broadcast_to(jnp.sum(xx), (L,)) * (1.0 / C)
                        for xx in s]
                ms2 = [jnp.broadcast_to(jnp.sum(xx), (L,)) * (1.0 / C)
                       for xx in sq]
                var = [m2 - mu * mu for m2, mu in zip(ms2, mean)]
                rstd = [_rsqrt(v + EPS) for v in var]
                mr = [mu * rs for mu, rs in zip(mean, rstd)]
                for i in range(RB):
                    for ci in range(C // L):
                        hn = h[i][ci] * rstd[i] - mr[i]
                        out_v[b, r0 + i, pl.ds(ci * L, L)] = (
                            hn * gv[ci] + bv[ci])
                return carry2

            lax.fori_loop(0, CHUNK // RB, blk_body, jnp.int32(0))

            pltpu.async_copy(out_v.at[b], out_hbm.at[base_b + c], wsem[b])

            @pl.when(c + NB < NCHUNK)
            def _prefetch():
                gather_descs(c + NB, b)
        return carry

    lax.fori_loop(0, PAIRS, pair_body, jnp.int32(0))

    for b in range(NB):
        write_desc(NCHUNK - NB + b, b).wait()


def _run(x1, temb, pemb, gamma, beta):
    mesh = plsc.VectorSubcoreMesh(
        core_axis_name="c", subcore_axis_name="s",
        num_cores=NC, num_subcores=NS)
    f = pl.kernel(
        _body,
        out_type=jax.ShapeDtypeStruct((B, T, C), jnp.float32),
        mesh=mesh,
        scratch_types=[
            pltpu.VMEM((ROWS_PER_W,), jnp.int32),
            pltpu.VMEM((NB, CHUNK, C), jnp.float32),
            pltpu.VMEM((NB, CHUNK, C), jnp.float32),
            pltpu.VMEM((T, C), jnp.float32),
            pltpu.VMEM((C,), jnp.float32),
            pltpu.VMEM((C,), jnp.float32),
            pltpu.SemaphoreType.DMA,
            pltpu.SemaphoreType.DMA,
            pltpu.SemaphoreType.DMA,
            pltpu.SemaphoreType.DMA,
        ],
        compiler_params=pltpu.CompilerParams(
            needs_layout_passes=False, use_tc_tiling_on_sc=False),
    )
    return f(x1, temb, pemb, gamma, beta)


@jax.jit
def _kernel_impl(x, temb, pemb, gamma, beta):
    return _run(x.reshape(N), temb, pemb, gamma, beta)


def kernel(x, temb, pemb, gamma, beta):
    return _kernel_impl(x, temb, pemb, gamma, beta)


# speedup vs baseline: 2.0020x; 1.0479x over previous
import jax
import jax.numpy as jnp
from jax import lax
from jax.experimental import pallas as pl
from jax.experimental.pallas import tpu as pltpu
from jax.experimental.pallas import tpu_sc as plsc
from jax._src.pallas.mosaic import sc_lowering as _sc_lowering
from jax._src.lib.mlir.dialects import math as _mlir_math

@_sc_lowering.register_lowering_rule(lax.rsqrt_p)
def _sc_rsqrt_rule(ctx, x, accuracy=None):
    del ctx, accuracy
    return _mlir_math.rsqrt(x)

B = 1024
T = 200
C = 64
N = B * T

NC = 2
NS = 16
NW = NC * NS
L = 16

B_PER_W = B // NW
ROWS_PER_W = N // NW
CHUNK = T
NCHUNK = B_PER_W
NB = 2
PAIRS = NCHUNK // NB
W0 = 128
W1 = CHUNK - W0

EPS = 1e-5


def _rsqrt(v):
    i = lax.bitcast_convert_type(v, jnp.int32)
    y = lax.bitcast_convert_type(
        jnp.int32(0x5F3759DF) - lax.shift_right_arithmetic(i, 1), jnp.float32)
    half_v = v * 0.5
    for _ in range(2):
        y = y * (1.5 - half_v * y * y)
    return y


def _body(x_hbm, temb_hbm, pemb_hbm, gamma_hbm, beta_hbm, out_hbm,
          idx_all, rows_v, out_v, pemb_v, g_v, b_v,
          gsem0, gsem1, wsem0, wsem1):
    gsem = (gsem0, gsem1)
    wsem = (wsem0, wsem1)
    wid = lax.axis_index("s") * NC + lax.axis_index("c")
    base = wid * ROWS_PER_W
    base_b = wid * B_PER_W

    pltpu.sync_copy(pemb_hbm, pemb_v)
    pltpu.sync_copy(gamma_hbm, g_v)
    pltpu.sync_copy(beta_hbm, b_v)
    pltpu.sync_copy(x_hbm.at[pl.ds(pl.multiple_of(base, ROWS_PER_W),
                                   ROWS_PER_W)], idx_all)

    gv = [g_v[pl.ds(c * L, L)] for c in range(C // L)]
    bv = [b_v[pl.ds(c * L, L)] for c in range(C // L)]

    def gather_descs(c, b, make_only=False):
        off = pl.multiple_of(c * CHUNK, 8)
        mk = pltpu.make_async_copy if make_only else pltpu.async_copy
        d0 = mk(temb_hbm.at[idx_all.at[pl.ds(off, W0)]],
                rows_v.at[b, pl.ds(0, W0)], gsem[b])
        d1 = mk(temb_hbm.at[idx_all.at[pl.ds(off + W0, W1)]],
                rows_v.at[b, pl.ds(W0, W1)], gsem[b])
        return d0, d1

    def write_desc(c, b):
        return pltpu.make_async_copy(out_v.at[b], out_hbm.at[base_b + c],
                                     wsem[b])

    gather_descs(0, 0)
    gather_descs(1, 1)

    def pair_body(m, carry):
        for b in range(NB):
            c = m * NB + b
            for d in gather_descs(c, b, make_only=True):
                d.wait()

            @pl.when(m > 0)
            def _wait_prev():
                write_desc(c - NB, b).wait()

            RB = 4

            def blk_body(mm, carry2):
                r0 = mm * RB
                h = [[rows_v[b, r0 + i, pl.ds(ci * L, L)]
                      + pemb_v[r0 + i, pl.ds(ci * L, L)]
                      for ci in range(C // L)]
                     for i in range(RB)]
                s = [(hh[0] + hh[1]) + (hh[2] + hh[3]) for hh in h]
                sq = [((hh[0] * hh[0] + hh[1] * hh[1])
                       + (hh[2] * hh[2] + hh[3] * hh[3])) for hh in h]
                mean = [jnp.broadcast_to(jnp.sum(xx), (L,)) * (1.0 / C)
                        for xx in s]
                ms2 = [jnp.broadcast_to(jnp.sum(xx), (L,)) * (1.0 / C)
                       for xx in sq]
                var = [m2 - mu * mu for m2, mu in zip(ms2, mean)]
                rstd = [lax.rsqrt(v + EPS) for v in var]
                mr = [mu * rs for mu, rs in zip(mean, rstd)]
                for i in range(RB):
                    for ci in range(C // L):
                        hn = h[i][ci] * rstd[i] - mr[i]
                        out_v[b, r0 + i, pl.ds(ci * L, L)] = (
                            hn * gv[ci] + bv[ci])
                return carry2

            lax.fori_loop(0, CHUNK // RB, blk_body, jnp.int32(0))

            pltpu.async_copy(out_v.at[b], out_hbm.at[base_b + c], wsem[b])

            @pl.when(c + NB < NCHUNK)
            def _prefetch():
                gather_descs(c + NB, b)
        return carry

    lax.fori_loop(0, PAIRS, pair_body, jnp.int32(0))

    for b in range(NB):
        write_desc(NCHUNK - NB + b, b).wait()


def _run(x1, temb, pemb, gamma, beta):
    mesh = plsc.VectorSubcoreMesh(
        core_axis_name="c", subcore_axis_name="s",
        num_cores=NC, num_subcores=NS)
    f = pl.kernel(
        _body,
        out_type=jax.ShapeDtypeStruct((B, T, C), jnp.float32),
        mesh=mesh,
        scratch_types=[
            pltpu.VMEM((ROWS_PER_W,), jnp.int32),
            pltpu.VMEM((NB, CHUNK, C), jnp.float32),
            pltpu.VMEM((NB, CHUNK, C), jnp.float32),
            pltpu.VMEM((T, C), jnp.float32),
            pltpu.VMEM((C,), jnp.float32),
            pltpu.VMEM((C,), jnp.float32),
            pltpu.SemaphoreType.DMA,
            pltpu.SemaphoreType.DMA,
            pltpu.SemaphoreType.DMA,
            pltpu.SemaphoreType.DMA,
        ],
        compiler_params=pltpu.CompilerParams(
            needs_layout_passes=False, use_tc_tiling_on_sc=False),
    )
    return f(x1, temb, pemb, gamma, beta)


@jax.jit
def _kernel_impl(x, temb, pemb, gamma, beta):
    return _run(x.reshape(N), temb, pemb, gamma, beta)


def kernel(x, temb, pemb, gamma, beta):
    return _kernel_impl(x, temb, pemb, gamma, beta)


# speedup vs baseline: 2.0032x; 1.0006x over previous
import jax
import jax.numpy as jnp
from jax import lax
from jax.experimental import pallas as pl
from jax.experimental.pallas import tpu as pltpu
from jax.experimental.pallas import tpu_sc as plsc
from jax.experimental import layout as jax_layout
from jax._src.pallas.mosaic import sc_lowering as _sc_lowering
from jax._src.lib.mlir.dialects import math as _mlir_math

@_sc_lowering.register_lowering_rule(lax.rsqrt_p)
def _sc_rsqrt_rule(ctx, x, accuracy=None):
    del ctx, accuracy
    return _mlir_math.rsqrt(x)

B = 1024
T = 200
C = 64
N = B * T

NC = 2
NS = 16
NW = NC * NS
L = 16

B_PER_W = B // NW
ROWS_PER_W = N // NW
CHUNK = T
NCHUNK = B_PER_W
NB = 2
PAIRS = NCHUNK // NB
W0 = 128
W1 = CHUNK - W0

EPS = 1e-5


def _rsqrt(v):
    i = lax.bitcast_convert_type(v, jnp.int32)
    y = lax.bitcast_convert_type(
        jnp.int32(0x5F3759DF) - lax.shift_right_arithmetic(i, 1), jnp.float32)
    half_v = v * 0.5
    for _ in range(2):
        y = y * (1.5 - half_v * y * y)
    return y


def _body(x_hbm, temb_hbm, pemb_hbm, gamma_hbm, beta_hbm, out_hbm,
          idx_all, rows_v, out_v, pemb_v, g_v, b_v,
          gsem0, gsem1, wsem0, wsem1):
    gsem = (gsem0, gsem1)
    wsem = (wsem0, wsem1)
    wid = lax.axis_index("s") * NC + lax.axis_index("c")
    base = wid * ROWS_PER_W
    base_b = wid * B_PER_W

    pltpu.sync_copy(pemb_hbm, pemb_v)
    pltpu.sync_copy(gamma_hbm, g_v)
    pltpu.sync_copy(beta_hbm, b_v)
    pltpu.sync_copy(x_hbm.at[pl.ds(pl.multiple_of(base, ROWS_PER_W),
                                   ROWS_PER_W)], idx_all)

    gv = [g_v[pl.ds(c * L, L)] for c in range(C // L)]
    bv = [b_v[pl.ds(c * L, L)] for c in range(C // L)]

    def gather_descs(c, b, make_only=False):
        off = pl.multiple_of(c * CHUNK, 8)
        mk = pltpu.make_async_copy if make_only else pltpu.async_copy
        d0 = mk(temb_hbm.at[idx_all.at[pl.ds(off, W0)]],
                rows_v.at[b, pl.ds(0, W0)], gsem[b])
        d1 = mk(temb_hbm.at[idx_all.at[pl.ds(off + W0, W1)]],
                rows_v.at[b, pl.ds(W0, W1)], gsem[b])
        return d0, d1

    def write_desc(c, b):
        return pltpu.make_async_copy(out_v.at[b], out_hbm.at[base_b + c],
                                     wsem[b])

    gather_descs(0, 0)
    gather_descs(1, 1)

    def pair_body(m, carry):
        for b in range(NB):
            c = m * NB + b
            for d in gather_descs(c, b, make_only=True):
                d.wait()

            @pl.when(m > 0)
            def _wait_prev():
                write_desc(c - NB, b).wait()

            RB = 4

            def blk_body(mm, carry2):
                r0 = mm * RB
                h = [[rows_v[b, r0 + i, pl.ds(ci * L, L)]
                      + pemb_v[r0 + i, pl.ds(ci * L, L)]
                      for ci in range(C // L)]
                     for i in range(RB)]
                s = [(hh[0] + hh[1]) + (hh[2] + hh[3]) for hh in h]
                sq = [((hh[0] * hh[0] + hh[1] * hh[1])
                       + (hh[2] * hh[2] + hh[3] * hh[3])) for hh in h]
                mean = [jnp.broadcast_to(jnp.sum(xx), (L,)) * (1.0 / C)
                        for xx in s]
                ms2 = [jnp.broadcast_to(jnp.sum(xx), (L,)) * (1.0 / C)
                       for xx in sq]
                var = [m2 - mu * mu for m2, mu in zip(ms2, mean)]
                rstd = [lax.rsqrt(v + EPS) for v in var]
                mr = [mu * rs for mu, rs in zip(mean, rstd)]
                for i in range(RB):
                    for ci in range(C // L):
                        hn = h[i][ci] * rstd[i] - mr[i]
                        out_v[b, r0 + i, pl.ds(ci * L, L)] = (
                            hn * gv[ci] + bv[ci])
                return carry2

            lax.fori_loop(0, CHUNK // RB, blk_body, jnp.int32(0))

            pltpu.async_copy(out_v.at[b], out_hbm.at[base_b + c], wsem[b])

            @pl.when(c + NB < NCHUNK)
            def _prefetch():
                gather_descs(c + NB, b)
        return carry

    lax.fori_loop(0, PAIRS, pair_body, jnp.int32(0))

    for b in range(NB):
        write_desc(NCHUNK - NB + b, b).wait()


def _run(x1, temb, pemb, gamma, beta):
    mesh = plsc.VectorSubcoreMesh(
        core_axis_name="c", subcore_axis_name="s",
        num_cores=NC, num_subcores=NS)
    f = pl.kernel(
        _body,
        out_type=jax.ShapeDtypeStruct((B, T, C), jnp.float32),
        mesh=mesh,
        scratch_types=[
            pltpu.VMEM((ROWS_PER_W,), jnp.int32),
            pltpu.VMEM((NB, CHUNK, C), jnp.float32),
            pltpu.VMEM((NB, CHUNK, C), jnp.float32),
            pltpu.VMEM((T, C), jnp.float32),
            pltpu.VMEM((C,), jnp.float32),
            pltpu.VMEM((C,), jnp.float32),
            pltpu.SemaphoreType.DMA,
            pltpu.SemaphoreType.DMA,
            pltpu.SemaphoreType.DMA,
            pltpu.SemaphoreType.DMA,
        ],
        compiler_params=pltpu.CompilerParams(
            needs_layout_passes=False, use_tc_tiling_on_sc=False),
    )
    return f(x1, temb, pemb, gamma, beta)


def _impl(x, temb, pemb, gamma, beta):
    return _run(x.reshape(N), temb, pemb, gamma, beta)


_cached_impl = None


def _get_impl():
    global _cached_impl
    if _cached_impl is None:
        try:
            dev = jax.devices()[0]
            if dev.platform != "tpu":
                raise ValueError("layout pinning needs a TPU device")
            fmt = jax_layout.Format(
                jax_layout.Layout(major_to_minor=(0, 1, 2), tiling=()),
                jax.sharding.SingleDeviceSharding(dev))
            _cached_impl = jax.jit(_impl, out_shardings=fmt)
        except Exception:
            _cached_impl = jax.jit(_impl)
    return _cached_impl


def kernel(x, temb, pemb, gamma, beta):
    return _get_impl()(x, temb, pemb, gamma, beta)
